# Initial kernel scaffold; baseline (speedup 1.0000x reference)
import jax
import jax.numpy as jnp
from jax.experimental import pallas as pl


def kernel(x):
    raise NotImplementedError("write your pallas kernel here")


# speedup vs baseline: 60.6801x; 60.6801x over previous
import numpy as np
import jax
import jax.numpy as jnp
from jax.experimental import pallas as pl

_NBINS = 10
_POOL = 8

_COS = tuple(0.0 if k * 2 == _NBINS else float(np.cos(k * np.pi / _NBINS))
             for k in range(1, _NBINS))
_SIN = tuple(1.0 if k * 2 == _NBINS else float(np.sin(k * np.pi / _NBINS))
             for k in range(1, _NBINS))


def _hog_cell(x_ref, hist_ref, px_ref):
    X = x_ref[0, 0]
    H, W = X.shape
    z_row = jnp.zeros((1, W), X.dtype)
    z_col = jnp.zeros((H, 1), X.dtype)

    Xb = X.astype(jnp.bfloat16).astype(jnp.float32)
    up = jnp.concatenate([z_row, Xb[:-1]], axis=0)
    dn = jnp.concatenate([Xb[1:], z_row], axis=0)
    lf = jnp.concatenate([z_col, Xb[:, :-1]], axis=1)
    rt = jnp.concatenate([Xb[:, 1:], z_col], axis=1)

    sv = up + 2.0 * Xb + dn
    sh = lf + 2.0 * Xb + rt
    g0 = (jnp.concatenate([z_col, sv[:, :-1]], axis=1)
          - jnp.concatenate([sv[:, 1:], z_col], axis=1))
    g1 = (jnp.concatenate([z_row, sh[:-1]], axis=0)
          - jnp.concatenate([sh[1:], z_row], axis=0))

    mag = jnp.sqrt(g0 * g0 + g1 * g1)
    neg = (g0 < 0.0) | ((g0 == 0.0) & (g1 < 0.0))
    gx = jnp.where(neg, -g0, g0)
    gy = jnp.where(neg, -g1, g1)
    axis0 = g0 == 0.0

    def rowpool(a):
        return a.reshape(H // _POOL, _POOL, W).sum(axis=1)

    one_minus = 1.0 - mag
    pa = [rowpool(mag)]
    pb = [float(_POOL) - pa[0]]
    for k in range(_NBINS - 1):
        cross = gx * _COS[k] - gy * _SIN[k]
        pa.append(rowpool(jnp.where(cross >= 0.0, mag, 0.0)))
        pb.append(rowpool(jnp.where((cross > 0.0) | axis0, one_minus, 0.0)))
    zero = jnp.zeros_like(pa[0])
    pa.append(zero)
    pb.append(zero)

    def colpool(a):
        t = a.T.reshape(W // _POOL, _POOL, H // _POOL).sum(axis=1)
        return t.T * (1.0 / (_POOL * _POOL))

    for b in range(_NBINS):
        kc = (b - 1) % _NBINS
        hb = (pa[b] - pa[b + 1]) + (pb[kc] - pb[kc + 1])
        hist_ref[0, b] = colpool(hb)
    px_ref[0, 0] = colpool(rowpool(X))


def kernel(x):
    n, c, h, w = x.shape
    hp, wp = h // _POOL, w // _POOL
    hist, px = pl.pallas_call(
        _hog_cell,
        grid=(n, c),
        in_specs=[pl.BlockSpec((1, 1, h, w), lambda i, j: (i, j, 0, 0))],
        out_specs=[
            pl.BlockSpec((1, _NBINS, hp, wp), lambda i, j: (i, j, 0, 0)),
            pl.BlockSpec((1, 1, hp, wp), lambda i, j: (i, j, 0, 0)),
        ],
        out_shape=[
            jax.ShapeDtypeStruct((n, c * _NBINS, hp, wp), x.dtype),
            jax.ShapeDtypeStruct((n, c, hp, wp), x.dtype),
        ],
    )(x)
    return jnp.concatenate([hist, px], axis=1)


# speedup vs baseline: 97.0720x; 1.5997x over previous
import numpy as np
import jax
import jax.numpy as jnp
from jax.experimental import pallas as pl

_NBINS = 10
_POOL = 8

_COT = tuple(0.0 if k * 2 == _NBINS else float(1.0 / np.tan(k * np.pi / _NBINS))
             for k in range(1, _NBINS))


def _hog_cell(x_ref, hist_ref, px_ref):
    Y = x_ref[0, 0]
    S, R, W = Y.shape
    z_row = jnp.zeros((1, W), Y.dtype)
    z_col = jnp.zeros((S, R, 1), Y.dtype)

    Yb = Y.astype(jnp.bfloat16).astype(jnp.float32)

    up7 = jnp.concatenate([z_row, Yb[S - 1, :-1]], axis=0)
    up = jnp.concatenate([up7[None], Yb[:-1]], axis=0)
    dn0 = jnp.concatenate([Yb[0, 1:], z_row], axis=0)
    dn = jnp.concatenate([Yb[1:], dn0[None]], axis=0)
    lf = jnp.concatenate([z_col, Yb[:, :, :-1]], axis=2)
    rt = jnp.concatenate([Yb[:, :, 1:], z_col], axis=2)

    sv = up + 2.0 * Yb + dn
    sh = lf + 2.0 * Yb + rt
    g0 = (jnp.concatenate([z_col, sv[:, :, :-1]], axis=2)
          - jnp.concatenate([sv[:, :, 1:], z_col], axis=2))
    sh_up7 = jnp.concatenate([z_row, sh[S - 1, :-1]], axis=0)
    sh_dn0 = jnp.concatenate([sh[0, 1:], z_row], axis=0)
    g1 = (jnp.concatenate([sh_up7[None], sh[:-1]], axis=0)
          - jnp.concatenate([sh[1:], sh_dn0[None]], axis=0))

    mag = jnp.sqrt(g0 * g0 + g1 * g1)
    neg = (g0 < 0.0) | ((g0 == 0.0) & (g1 < 0.0))
    gx = jnp.where(neg, -g0, g0)
    gy = jnp.where(neg, -g1, g1)
    axis0 = g0 == 0.0

    def rowpool(a):
        return a.sum(axis=0)

    one_minus = 1.0 - mag
    pa = [rowpool(mag)]
    pb = [float(_POOL) - pa[0]]
    for k in range(_NBINS - 1):
        cross = gx * _COT[k] - gy
        pa.append(rowpool(jnp.where(cross >= 0.0, mag, 0.0)))
        pb.append(rowpool(jnp.where((cross > 0.0) | axis0, one_minus, 0.0)))
    zero = jnp.zeros_like(pa[0])
    pa.append(zero)
    pb.append(zero)

    chans = []
    for b in range(_NBINS):
        kc = (b - 1) % _NBINS
        chans.append((pa[b] - pa[b + 1]) + (pb[kc] - pb[kc + 1]))
    chans.append(rowpool(Y))
    C = jnp.concatenate(chans, axis=0)

    jj = jax.lax.broadcasted_iota(jnp.int32, (W, W // _POOL), 0)
    cc = jax.lax.broadcasted_iota(jnp.int32, (W, W // _POOL), 1)
    P = (jj // _POOL == cc).astype(jnp.bfloat16)
    hi = C.astype(jnp.bfloat16)
    lo = (C - hi.astype(jnp.float32)).astype(jnp.bfloat16)
    res = (jnp.dot(hi, P, preferred_element_type=jnp.float32)
           + jnp.dot(lo, P, preferred_element_type=jnp.float32))
    res = res * (1.0 / (_POOL * _POOL))

    for b in range(_NBINS):
        hist_ref[0, b] = res[b * 64:(b + 1) * 64]
    px_ref[0, 0] = res[_NBINS * 64:(_NBINS + 1) * 64]


def kernel(x):
    n, c, h, w = x.shape
    hp, wp = h // _POOL, w // _POOL
    xs = x.reshape(n, c, hp, _POOL, w).transpose(0, 1, 3, 2, 4)
    hist, px = pl.pallas_call(
        _hog_cell,
        grid=(n, c),
        in_specs=[pl.BlockSpec((1, 1, _POOL, hp, w), lambda i, j: (i, j, 0, 0, 0))],
        out_specs=[
            pl.BlockSpec((1, _NBINS, hp, wp), lambda i, j: (i, j, 0, 0)),
            pl.BlockSpec((1, 1, hp, wp), lambda i, j: (i, j, 0, 0)),
        ],
        out_shape=[
            jax.ShapeDtypeStruct((n, c * _NBINS, hp, wp), x.dtype),
            jax.ShapeDtypeStruct((n, c, hp, wp), x.dtype),
        ],
    )(xs)
    return jnp.concatenate([hist, px], axis=1)


# speedup vs baseline: 100.4370x; 1.0347x over previous
import numpy as np
import jax
import jax.numpy as jnp
from jax.experimental import pallas as pl
from jax.experimental.pallas import tpu as pltpu

_NBINS = 10
_POOL = 8

_COT = tuple(0.0 if k * 2 == _NBINS else float(1.0 / np.tan(k * np.pi / _NBINS))
             for k in range(1, _NBINS))


def _hog_cell(x_hbm, hist_ref, px_ref, scr, sem):
    S, W = _POOL, 512
    z_row = jnp.zeros((1, W), jnp.float32)
    z_col = jnp.zeros((W // S, 1), jnp.float32)

    i = pl.program_id(0)
    j = pl.program_id(1)
    copies = [pltpu.make_async_copy(x_hbm.at[i, j, :, s, :], scr.at[s], sem)
              for s in range(S)]
    for c in copies:
        c.start()
    for c in copies:
        c.wait()
    Y = [scr[s] for s in range(S)]
    Yb = [y.astype(jnp.bfloat16).astype(jnp.float32) for y in Y]

    up = [jnp.concatenate([z_row, Yb[S - 1][:-1]], axis=0)] + Yb[:-1]
    dn = Yb[1:] + [jnp.concatenate([Yb[0][1:], z_row], axis=0)]
    lf = [jnp.concatenate([z_col, y[:, :-1]], axis=1) for y in Yb]
    rt = [jnp.concatenate([y[:, 1:], z_col], axis=1) for y in Yb]

    sv = [up[s] + 2.0 * Yb[s] + dn[s] for s in range(S)]
    sh = [lf[s] + 2.0 * Yb[s] + rt[s] for s in range(S)]
    g0 = [jnp.concatenate([z_col, sv[s][:, :-1]], axis=1)
          - jnp.concatenate([sv[s][:, 1:], z_col], axis=1) for s in range(S)]
    sh_up = [jnp.concatenate([z_row, sh[S - 1][:-1]], axis=0)] + sh[:-1]
    sh_dn = sh[1:] + [jnp.concatenate([sh[0][1:], z_row], axis=0)]
    g1 = [sh_up[s] - sh_dn[s] for s in range(S)]

    mag = [jnp.sqrt(g0[s] * g0[s] + g1[s] * g1[s]) for s in range(S)]
    neg = [(g0[s] < 0.0) | ((g0[s] == 0.0) & (g1[s] < 0.0)) for s in range(S)]
    gx = [jnp.where(neg[s], -g0[s], g0[s]) for s in range(S)]
    gy = [jnp.where(neg[s], -g1[s], g1[s]) for s in range(S)]
    axis0 = [g0[s] == 0.0 for s in range(S)]

    def accum(parts):
        t = parts[0]
        for p in parts[1:]:
            t = t + p
        return t

    om = [1.0 - mag[s] for s in range(S)]
    pa = [accum(mag)]
    pb = [float(_POOL) - pa[0]]
    for k in range(_NBINS - 1):
        cross = [gx[s] * _COT[k] - gy[s] for s in range(S)]
        pa.append(accum([jnp.where(cross[s] >= 0.0, mag[s], 0.0)
                         for s in range(S)]))
        pb.append(accum([jnp.where((cross[s] > 0.0) | axis0[s], om[s], 0.0)
                         for s in range(S)]))
    zero = jnp.zeros_like(pa[0])
    pa.append(zero)
    pb.append(zero)

    chans = []
    for b in range(_NBINS):
        kc = (b - 1) % _NBINS
        chans.append((pa[b] - pa[b + 1]) + (pb[kc] - pb[kc + 1]))
    chans.append(accum(Y))
    C = jnp.concatenate(chans, axis=0)

    jj = jax.lax.broadcasted_iota(jnp.int32, (W, W // _POOL), 0)
    cc = jax.lax.broadcasted_iota(jnp.int32, (W, W // _POOL), 1)
    P = (jj // _POOL == cc).astype(jnp.bfloat16)
    hi = C.astype(jnp.bfloat16)
    lo = (C - hi.astype(jnp.float32)).astype(jnp.bfloat16)
    res = (jnp.dot(hi, P, preferred_element_type=jnp.float32)
           + jnp.dot(lo, P, preferred_element_type=jnp.float32))
    res = res * (1.0 / (_POOL * _POOL))

    for b in range(_NBINS):
        hist_ref[0, b] = res[b * 64:(b + 1) * 64]
    px_ref[0, 0] = res[_NBINS * 64:(_NBINS + 1) * 64]


def kernel(x):
    n, c, h, w = x.shape
    hp, wp = h // _POOL, w // _POOL
    xs = x.reshape(n, c, hp, _POOL, w)
    hist, px = pl.pallas_call(
        _hog_cell,
        grid=(n, c),
        in_specs=[pl.BlockSpec(memory_space=pl.ANY)],
        scratch_shapes=[
            pltpu.VMEM((_POOL, hp, w), jnp.float32),
            pltpu.SemaphoreType.DMA,
        ],
        out_specs=[
            pl.BlockSpec((1, _NBINS, hp, wp), lambda i, j: (i, j, 0, 0)),
            pl.BlockSpec((1, 1, hp, wp), lambda i, j: (i, j, 0, 0)),
        ],
        out_shape=[
            jax.ShapeDtypeStruct((n, c * _NBINS, hp, wp), x.dtype),
            jax.ShapeDtypeStruct((n, c, hp, wp), x.dtype),
        ],
    )(xs)
    return jnp.concatenate([hist, px], axis=1)


# speedup vs baseline: 125.2626x; 1.2472x over previous
import numpy as np
import jax
import jax.numpy as jnp
from jax.experimental import pallas as pl
from jax.experimental.pallas import tpu as pltpu

_NBINS = 10
_POOL = 8

_COT = tuple(0.0 if k * 2 == _NBINS else float(1.0 / np.tan(k * np.pi / _NBINS))
             for k in range(1, _NBINS))


def _hog_cell(x_hbm, hist_ref, px_ref, scr, sem):
    S, W = _POOL, 512
    z_row = jnp.zeros((1, W), jnp.float32)
    z_col = jnp.zeros((W // S, 1), jnp.float32)

    i = pl.program_id(0)
    j = pl.program_id(1)
    nj = pl.num_programs(1)
    t = i * nj + j
    buf = jax.lax.rem(t, 2)

    def slab_copies(ci, cj, b):
        return [pltpu.make_async_copy(x_hbm.at[ci, cj, :, s, :],
                                      scr.at[b, s], sem.at[b])
                for s in range(S)]

    @pl.when(t == 0)
    def _prime():
        for c in slab_copies(i, j, buf):
            c.start()

    @pl.when(t + 1 < pl.num_programs(0) * nj)
    def _prefetch():
        jn = jnp.where(j + 1 == nj, 0, j + 1)
        in_ = jnp.where(j + 1 == nj, i + 1, i)
        for c in slab_copies(in_, jn, 1 - buf):
            c.start()

    for c in slab_copies(i, j, buf):
        c.wait()
    Y = [scr[buf, s] for s in range(S)]
    Yb = [y.astype(jnp.bfloat16).astype(jnp.float32) for y in Y]

    up = [jnp.concatenate([z_row, Yb[S - 1][:-1]], axis=0)] + Yb[:-1]
    dn = Yb[1:] + [jnp.concatenate([Yb[0][1:], z_row], axis=0)]
    lf = [jnp.concatenate([z_col, y[:, :-1]], axis=1) for y in Yb]
    rt = [jnp.concatenate([y[:, 1:], z_col], axis=1) for y in Yb]

    sv = [up[s] + 2.0 * Yb[s] + dn[s] for s in range(S)]
    sh = [lf[s] + 2.0 * Yb[s] + rt[s] for s in range(S)]
    g0 = [jnp.concatenate([z_col, sv[s][:, :-1]], axis=1)
          - jnp.concatenate([sv[s][:, 1:], z_col], axis=1) for s in range(S)]
    sh_up = [jnp.concatenate([z_row, sh[S - 1][:-1]], axis=0)] + sh[:-1]
    sh_dn = sh[1:] + [jnp.concatenate([sh[0][1:], z_row], axis=0)]
    g1 = [sh_up[s] - sh_dn[s] for s in range(S)]

    mag = [jnp.sqrt(g0[s] * g0[s] + g1[s] * g1[s]) for s in range(S)]
    neg = [(g0[s] < 0.0) | ((g0[s] == 0.0) & (g1[s] < 0.0)) for s in range(S)]
    gx = [jnp.where(neg[s], -g0[s], g0[s]) for s in range(S)]
    gy = [jnp.where(neg[s], -g1[s], g1[s]) for s in range(S)]
    axis0 = [g0[s] == 0.0 for s in range(S)]

    def accum(parts):
        t = parts[0]
        for p in parts[1:]:
            t = t + p
        return t

    om = [1.0 - mag[s] for s in range(S)]
    pa = [accum(mag)]
    pb = [float(_POOL) - pa[0]]
    for k in range(_NBINS - 1):
        cross = [gx[s] * _COT[k] - gy[s] for s in range(S)]
        pa.append(accum([jnp.where(cross[s] >= 0.0, mag[s], 0.0)
                         for s in range(S)]))
        pb.append(accum([jnp.where((cross[s] > 0.0) | axis0[s], om[s], 0.0)
                         for s in range(S)]))
    zero = jnp.zeros_like(pa[0])
    pa.append(zero)
    pb.append(zero)

    chans = []
    for b in range(_NBINS):
        kc = (b - 1) % _NBINS
        chans.append((pa[b] - pa[b + 1]) + (pb[kc] - pb[kc + 1]))
    chans.append(accum(Y))
    C = jnp.concatenate(chans, axis=0)

    jj = jax.lax.broadcasted_iota(jnp.int32, (W, W // _POOL), 0)
    cc = jax.lax.broadcasted_iota(jnp.int32, (W, W // _POOL), 1)
    P = (jj // _POOL == cc).astype(jnp.bfloat16)
    hi = C.astype(jnp.bfloat16)
    lo = (C - hi.astype(jnp.float32)).astype(jnp.bfloat16)
    res = (jnp.dot(hi, P, preferred_element_type=jnp.float32)
           + jnp.dot(lo, P, preferred_element_type=jnp.float32))
    res = res * (1.0 / (_POOL * _POOL))

    for b in range(_NBINS):
        hist_ref[0, b] = res[b * 64:(b + 1) * 64]
    px_ref[0, 0] = res[_NBINS * 64:(_NBINS + 1) * 64]


def kernel(x):
    n, c, h, w = x.shape
    hp, wp = h // _POOL, w // _POOL
    xs = x.reshape(n, c, hp, _POOL, w)
    hist, px = pl.pallas_call(
        _hog_cell,
        grid=(n, c),
        in_specs=[pl.BlockSpec(memory_space=pl.ANY)],
        scratch_shapes=[
            pltpu.VMEM((2, _POOL, hp, w), jnp.float32),
            pltpu.SemaphoreType.DMA((2,)),
        ],
        out_specs=[
            pl.BlockSpec((1, _NBINS, hp, wp), lambda i, j: (i, j, 0, 0)),
            pl.BlockSpec((1, 1, hp, wp), lambda i, j: (i, j, 0, 0)),
        ],
        out_shape=[
            jax.ShapeDtypeStruct((n, c * _NBINS, hp, wp), x.dtype),
            jax.ShapeDtypeStruct((n, c, hp, wp), x.dtype),
        ],
    )(xs)
    return jnp.concatenate([hist, px], axis=1)
